# baseline (speedup 1.0000x reference)
import jax
import jax.numpy as jnp
from jax.experimental import pallas as pl


def _sq_diff_sum(a_ref, b_ref, o_ref):
    d = a_ref[...] - b_ref[...]
    o_ref[...] = jnp.full((1, 1, 128), jnp.sum(d * d), jnp.float32)


def kernel(pc1, pc2):
    B = pc1.shape[0]
    n = pc1.shape[1] * pc1.shape[2]
    a = jnp.sort(pc1.reshape(B, -1), axis=1).reshape(B, n // 128, 128)
    b = jnp.sort(pc2.reshape(B, -1), axis=1).reshape(B, n // 128, 128)
    ss = pl.pallas_call(
        _sq_diff_sum,
        grid=(B,),
        in_specs=[
            pl.BlockSpec((1, n // 128, 128), lambda i: (i, 0, 0)),
            pl.BlockSpec((1, n // 128, 128), lambda i: (i, 0, 0)),
        ],
        out_specs=pl.BlockSpec((1, 1, 128), lambda i: (i, 0, 0)),
        out_shape=jax.ShapeDtypeStruct((B, 1, 128), jnp.float32),
    )(a, b)
    return jnp.mean(jnp.sqrt(ss[:, 0, 0]))
